# Initial kernel scaffold; baseline (speedup 1.0000x reference)
import jax
import jax.numpy as jnp
from jax.experimental import pallas as pl


def kernel(h, edge_weight, Wq, bq, Wk, bk, Wv, bv, Wo, bo, src_idx):
    raise NotImplementedError("write your pallas kernel here")


# speedup vs baseline: 6.6385x; 6.6385x over previous
import functools
import math

import jax
import jax.numpy as jnp
from jax import lax
from jax.experimental import pallas as pl
from jax.experimental.pallas import tpu as pltpu
from jax.experimental.pallas import tpu_sc as plsc

N = 10000
D = 32
HID = 128
H = 8
DH = HID // H
NK = 16
INV_SQRT_DH = 1.0 / math.sqrt(DH)

NC = 2
NS = 16
NW = NC * NS
NPAD = 10240
NPW = NPAD // NW
CH = 4
NCHUNK = NPW // CH

BLK = 512


def _qkv_ew_body(h_ref, ew_ref, wq_ref, bq_ref, wk_ref, bk_ref,
                 wv_ref, bv_ref, q_ref, k_ref, v_ref, ewn_ref):
    hb = h_ref[...]
    q_ref[...] = jnp.dot(hb, wq_ref[...],
                         preferred_element_type=jnp.float32) + bq_ref[...]
    k_ref[...] = jnp.dot(hb, wk_ref[...],
                         preferred_element_type=jnp.float32) + bk_ref[...]
    v_ref[...] = jnp.dot(hb, wv_ref[...],
                         preferred_element_type=jnp.float32) + bv_ref[...]
    w = ew_ref[...]
    col = lax.broadcasted_iota(jnp.int32, w.shape, 1)
    rank = jnp.zeros(w.shape, jnp.float32)
    for j in range(D):
        wj = w[:, j:j + 1]
        gt = wj > w
        eq = jnp.logical_and(wj == w, col > j)
        rank += jnp.logical_or(gt, eq).astype(jnp.float32)
    ew = w * (rank < NK).astype(jnp.float32)
    ewn_ref[...] = ew / (jnp.sum(ew, axis=1, keepdims=True) + 1e-5) \
        * INV_SQRT_DH


def _out_proj_body(x_ref, wo_ref, bo_ref, o_ref):
    y = jnp.dot(x_ref[...], wo_ref[...],
                preferred_element_type=jnp.float32) + bo_ref[...]
    o_ref[...] = jnp.where(y >= 0.0, y, 0.01 * y)


def _attn_sc_body(q_hbm, k_hbm, v_hbm, ew_hbm, si_hbm, out_hbm,
                  idx_v, krows, vrows, qrows, ewrows, orows, gsem):
    wid = lax.axis_index("s") * NC + lax.axis_index("c")
    lane = lax.iota(jnp.int32, 16)
    lane_row = lane * HID
    zeros16 = jnp.zeros((16,), jnp.float32)

    def chunk_body(g, carry):
        base = wid * NPW + g * CH
        pltpu.sync_copy(si_hbm.at[pl.ds(base * D, CH * D)], idx_v)
        pltpu.sync_copy(q_hbm.at[pl.ds(base, CH)], qrows)
        pltpu.sync_copy(ew_hbm.at[pl.ds(base, CH)], ewrows)
        cp_k = pltpu.async_copy(k_hbm.at[idx_v], krows, gsem)
        cp_v = pltpu.async_copy(v_hbm.at[idx_v], vrows, gsem)
        cp_k.wait()
        cp_v.wait()

        def node_body(nl, carry2):
            row0 = nl * D
            ew0 = ewrows[nl, pl.ds(0, 16)]
            ew1 = ewrows[nl, pl.ds(16, 16)]
            attn = []
            for h in range(H):
                qv = qrows[nl, pl.ds(h * DH, DH)]
                s0a = zeros16
                s0b = zeros16
                s1a = zeros16
                s1b = zeros16
                for dh in range(DH):
                    qs = qv[dh]
                    c = jnp.full((16,), h * DH + dh, jnp.int32)
                    g0 = plsc.load_gather(krows, [row0 + lane, c])
                    g1 = plsc.load_gather(krows, [row0 + 16 + lane, c])
                    if dh % 2 == 0:
                        s0a = s0a + g0 * qs
                        s1a = s1a + g1 * qs
                    else:
                        s0b = s0b + g0 * qs
                        s1b = s1b + g1 * qs
                l0 = (s0a + s0b) * ew0
                l1 = (s1a + s1b) * ew1
                m = jnp.maximum(jnp.max(l0), jnp.max(l1))
                p0 = jnp.exp(l0 - m)
                p1 = jnp.exp(l1 - m)
                z = jnp.sum(p0) + jnp.sum(p1)
                rz = jnp.full((16,), 1.0, jnp.float32) / (zeros16 + z)
                attn.append((p0 * rz, p1 * rz))
            accs = [zeros16 for _ in range(H)]
            for d in range(16):
                for h in range(H):
                    a0, a1 = attn[h]
                    vv0 = vrows[row0 + d, pl.ds(h * DH, DH)]
                    vv1 = vrows[row0 + 16 + d, pl.ds(h * DH, DH)]
                    accs[h] = accs[h] + a0[d] * vv0 + a1[d] * vv1
            for h in range(H):
                orows[nl, pl.ds(h * DH, DH)] = accs[h]
            return carry2

        lax.fori_loop(0, CH, node_body, 0)
        pltpu.sync_copy(orows, out_hbm.at[pl.ds(base, CH)])
        return carry

    lax.fori_loop(0, NCHUNK, chunk_body, 0)


_attn_sc = functools.partial(
    pl.kernel,
    out_type=jax.ShapeDtypeStruct((NPAD, HID), jnp.float32),
    mesh=plsc.VectorSubcoreMesh(core_axis_name="c", subcore_axis_name="s"),
    scratch_types=[
        pltpu.VMEM((CH * D,), jnp.int32),
        pltpu.VMEM((CH * D, HID), jnp.float32),
        pltpu.VMEM((CH * D, HID), jnp.float32),
        pltpu.VMEM((CH, HID), jnp.float32),
        pltpu.VMEM((CH, D), jnp.float32),
        pltpu.VMEM((CH, HID), jnp.float32),
        pltpu.SemaphoreType.DMA,
    ],
    compiler_params=pltpu.CompilerParams(needs_layout_passes=False),
)(_attn_sc_body)


def kernel(h, edge_weight, Wq, bq, Wk, bk, Wv, bv, Wo, bo, src_idx):
    hp = jnp.pad(h, ((0, NPAD - N), (0, 0)))
    ewp = jnp.pad(edge_weight[:, :, 0], ((0, NPAD - N), (0, 0)))
    sip = jnp.pad(src_idx, ((0, NPAD - N), (0, 0))).reshape(-1)

    grid = NPAD // BLK
    row_blk = pl.BlockSpec((BLK, HID), lambda i: (i, 0))
    ew_blk = pl.BlockSpec((BLK, D), lambda i: (i, 0))
    full = pl.BlockSpec((HID, HID), lambda i: (0, 0))
    bias = pl.BlockSpec((1, HID), lambda i: (0, 0))

    q, k, v, ewn = pl.pallas_call(
        _qkv_ew_body,
        grid=(grid,),
        in_specs=[row_blk, ew_blk, full, bias, full, bias, full, bias],
        out_specs=[row_blk, row_blk, row_blk, ew_blk],
        out_shape=[
            jax.ShapeDtypeStruct((NPAD, HID), jnp.float32),
            jax.ShapeDtypeStruct((NPAD, HID), jnp.float32),
            jax.ShapeDtypeStruct((NPAD, HID), jnp.float32),
            jax.ShapeDtypeStruct((NPAD, D), jnp.float32),
        ],
    )(hp, ewp, Wq, bq.reshape(1, HID), Wk, bk.reshape(1, HID),
      Wv, bv.reshape(1, HID))

    h_agg = _attn_sc(q, k, v, ewn, sip)

    out = pl.pallas_call(
        _out_proj_body,
        grid=(grid,),
        in_specs=[row_blk, full, bias],
        out_specs=row_blk,
        out_shape=jax.ShapeDtypeStruct((NPAD, HID), jnp.float32),
    )(h_agg, Wo, bo.reshape(1, HID))
    return out[:N]


# speedup vs baseline: 11.7113x; 1.7641x over previous
import functools
import math

import jax
import jax.numpy as jnp
from jax import lax
from jax.experimental import pallas as pl
from jax.experimental.pallas import tpu as pltpu
from jax.experimental.pallas import tpu_sc as plsc

N = 10000
D = 32
HID = 128
H = 8
DH = HID // H
NK = 16
INV_SQRT_DH = 1.0 / math.sqrt(DH)

NC = 2
NS = 16
NW = NC * NS
NPAD = 10240
NPW = NPAD // NW
CH = 4
NCHUNK = NPW // CH

BLK = 512


def _qkv_ew_body(h_ref, ew_ref, wq_ref, bq_ref, wk_ref, bk_ref,
                 wv_ref, bv_ref, q_ref, k_ref, v_ref, ewn_ref):
    hb = h_ref[...]
    q_ref[...] = jnp.dot(hb, wq_ref[...],
                         preferred_element_type=jnp.float32) + bq_ref[...]
    k_ref[...] = jnp.dot(hb, wk_ref[...],
                         preferred_element_type=jnp.float32) + bk_ref[...]
    v_ref[...] = jnp.dot(hb, wv_ref[...],
                         preferred_element_type=jnp.float32) + bv_ref[...]
    w = ew_ref[...]
    col = lax.broadcasted_iota(jnp.int32, w.shape, 1)
    rank = jnp.zeros(w.shape, jnp.float32)
    for j in range(D):
        wj = w[:, j:j + 1]
        gt = wj > w
        eq = jnp.logical_and(wj == w, col > j)
        rank += jnp.logical_or(gt, eq).astype(jnp.float32)
    ew = w * (rank < NK).astype(jnp.float32)
    ewn_ref[...] = ew / (jnp.sum(ew, axis=1, keepdims=True) + 1e-5) \
        * INV_SQRT_DH


def _out_proj_body(x_ref, wo_ref, bo_ref, o_ref):
    y = jnp.dot(x_ref[...], wo_ref[...],
                preferred_element_type=jnp.float32) + bo_ref[...]
    o_ref[...] = jnp.where(y >= 0.0, y, 0.01 * y)


def _attn_sc_body(q_hbm, k_hbm, v_hbm, ew_hbm, si_hbm, out_hbm,
                  idx_all, ew_all, qr, kr, vr, orows, gsem, osem):
    wid = lax.axis_index("s") * NC + lax.axis_index("c")
    base_w = wid * NPW
    lane = lax.iota(jnp.int32, 16)
    zeros16 = jnp.zeros((16,), jnp.float32)

    pltpu.sync_copy(si_hbm.at[pl.ds(base_w * D, NPW * D)], idx_all)
    pltpu.sync_copy(ew_hbm.at[pl.ds(base_w, NPW)], ew_all)

    def issue_gathers(g, s):
        gath = idx_all.at[pl.ds(g * CH * D, CH * D)]
        pltpu.async_copy(k_hbm.at[gath], kr[s], gsem[s])
        pltpu.async_copy(v_hbm.at[gath], vr[s], gsem[s])
        pltpu.async_copy(q_hbm.at[pl.ds(base_w + g * CH, CH)], qr[s], gsem[s])

    def wait_gathers(g, s):
        gath = idx_all.at[pl.ds(g * CH * D, CH * D)]
        pltpu.make_async_copy(k_hbm.at[gath], kr[s], gsem[s]).wait()
        pltpu.make_async_copy(v_hbm.at[gath], vr[s], gsem[s]).wait()
        pltpu.make_async_copy(
            q_hbm.at[pl.ds(base_w + g * CH, CH)], qr[s], gsem[s]).wait()

    def issue_out(g, s):
        pltpu.async_copy(
            orows[s], out_hbm.at[pl.ds(base_w + g * CH, CH)], osem[s])

    def wait_out(g, s):
        pltpu.make_async_copy(
            orows[s], out_hbm.at[pl.ds(base_w + g * CH, CH)],
            osem[s]).wait()

    def compute_chunk(g, s):
        krows = kr[s]
        vrows = vr[s]
        gq = g * CH

        def node_body(nl, carry2):
            row0 = nl * D
            ew0 = ew_all[gq + nl, pl.ds(0, 16)]
            ew1 = ew_all[gq + nl, pl.ds(16, 16)]
            attn = []
            for h in range(H):
                qv = qr[s][nl, pl.ds(h * DH, DH)]
                s0a = zeros16
                s0b = zeros16
                s1a = zeros16
                s1b = zeros16
                for dh in range(DH):
                    qs = qv[dh]
                    c = jnp.full((16,), h * DH + dh, jnp.int32)
                    g0 = plsc.load_gather(krows, [row0 + lane, c])
                    g1 = plsc.load_gather(krows, [row0 + 16 + lane, c])
                    if dh % 2 == 0:
                        s0a = s0a + g0 * qs
                        s1a = s1a + g1 * qs
                    else:
                        s0b = s0b + g0 * qs
                        s1b = s1b + g1 * qs
                l0 = (s0a + s0b) * ew0
                l1 = (s1a + s1b) * ew1
                m = jnp.maximum(jnp.max(l0), jnp.max(l1))
                p0 = jnp.exp(l0 - m)
                p1 = jnp.exp(l1 - m)
                z = jnp.sum(p0) + jnp.sum(p1)
                rz = jnp.full((16,), 1.0, jnp.float32) / (zeros16 + z)
                attn.append((p0 * rz, p1 * rz))
            accs = [zeros16 for _ in range(H)]
            for d in range(16):
                for h in range(H):
                    a0, a1 = attn[h]
                    vv0 = vrows[row0 + d, pl.ds(h * DH, DH)]
                    vv1 = vrows[row0 + 16 + d, pl.ds(h * DH, DH)]
                    accs[h] = accs[h] + a0[d] * vv0 + a1[d] * vv1
            for h in range(H):
                orows[s][nl, pl.ds(h * DH, DH)] = accs[h]
            return carry2

        lax.fori_loop(0, CH, node_body, 0)

    issue_gathers(0, 0)
    issue_gathers(1, 1)

    def pipe_body(g2, carry):
        for par in range(2):
            g = g2 + par

            @pl.when(g2 > 0)
            def _():
                wait_out(g - 2, par)

            wait_gathers(g, par)
            compute_chunk(g, par)
            issue_out(g, par)

            @pl.when(g + 2 < NCHUNK)
            def _():
                issue_gathers(g + 2, par)

        return carry

    lax.fori_loop(0, NCHUNK // 2, lambda i, c: pipe_body(i * 2, c), 0)
    wait_out(NCHUNK - 2, 0)
    wait_out(NCHUNK - 1, 1)


_attn_sc = functools.partial(
    pl.kernel,
    out_type=jax.ShapeDtypeStruct((NPAD, HID), jnp.float32),
    mesh=plsc.VectorSubcoreMesh(core_axis_name="c", subcore_axis_name="s"),
    scratch_types=[
        pltpu.VMEM((NPW * D,), jnp.int32),
        pltpu.VMEM((NPW, D), jnp.float32),
        [pltpu.VMEM((CH, HID), jnp.float32) for _ in range(2)],
        [pltpu.VMEM((CH * D, HID), jnp.float32) for _ in range(2)],
        [pltpu.VMEM((CH * D, HID), jnp.float32) for _ in range(2)],
        [pltpu.VMEM((CH, HID), jnp.float32) for _ in range(2)],
        [pltpu.SemaphoreType.DMA for _ in range(2)],
        [pltpu.SemaphoreType.DMA for _ in range(2)],
    ],
    compiler_params=pltpu.CompilerParams(needs_layout_passes=False),
)(_attn_sc_body)


def kernel(h, edge_weight, Wq, bq, Wk, bk, Wv, bv, Wo, bo, src_idx):
    hp = jnp.pad(h, ((0, NPAD - N), (0, 0)))
    ewp = jnp.pad(edge_weight[:, :, 0], ((0, NPAD - N), (0, 0)))
    sip = jnp.pad(src_idx, ((0, NPAD - N), (0, 0))).reshape(-1)

    grid = NPAD // BLK
    row_blk = pl.BlockSpec((BLK, HID), lambda i: (i, 0))
    ew_blk = pl.BlockSpec((BLK, D), lambda i: (i, 0))
    full = pl.BlockSpec((HID, HID), lambda i: (0, 0))
    bias = pl.BlockSpec((1, HID), lambda i: (0, 0))

    q, k, v, ewn = pl.pallas_call(
        _qkv_ew_body,
        grid=(grid,),
        in_specs=[row_blk, ew_blk, full, bias, full, bias, full, bias],
        out_specs=[row_blk, row_blk, row_blk, ew_blk],
        out_shape=[
            jax.ShapeDtypeStruct((NPAD, HID), jnp.float32),
            jax.ShapeDtypeStruct((NPAD, HID), jnp.float32),
            jax.ShapeDtypeStruct((NPAD, HID), jnp.float32),
            jax.ShapeDtypeStruct((NPAD, D), jnp.float32),
        ],
    )(hp, ewp, Wq, bq.reshape(1, HID), Wk, bk.reshape(1, HID),
      Wv, bv.reshape(1, HID))

    h_agg = _attn_sc(q, k, v, ewn, sip)

    out = pl.pallas_call(
        _out_proj_body,
        grid=(grid,),
        in_specs=[row_blk, full, bias],
        out_specs=row_blk,
        out_shape=jax.ShapeDtypeStruct((NPAD, HID), jnp.float32),
    )(h_agg, Wo, bo.reshape(1, HID))
    return out[:N]
